# baseline (speedup 1.0000x reference)
import functools

import jax
import jax.numpy as jnp
from jax import lax
from jax.experimental import pallas as pl
from jax.experimental.pallas import tpu as pltpu
from jax.experimental.pallas import tpu_sc as plsc

N = 10000
E = 320000
D = 128
NC = 2
NS = 16
NW = NC * NS
K = 80
EPT = E // NW
NCH = EPT // K
KA = 80
NCHA = 126
EPTA = NCHA * KA
NP = N + 8
ROWS_PT = N // NS

_mesh = plsc.VectorSubcoreMesh(core_axis_name="c", subcore_axis_name="s")


def _flush(acc, out_hbm, c, s):
    fbase = s * 624
    pltpu.sync_copy(acc.at[pl.ds(fbase, 624), :],
                    out_hbm.at[c, pl.ds(fbase, 624), :])

    @pl.when(s == NS - 1)
    def _():
        pltpu.sync_copy(acc.at[pl.ds(9984, 16), :],
                        out_hbm.at[c, pl.ds(9984, 16), :])


@functools.partial(
    pl.kernel,
    out_type=jax.ShapeDtypeStruct((NC, N, D), jnp.float32),
    mesh=_mesh,
    scratch_types=[
        pltpu.VMEM((NCH, K), jnp.int32),
        pltpu.VMEM((K, D), jnp.float32),
        pltpu.VMEM_SHARED((N, D), jnp.float32),
    ],
)
def _deg_kernel(dst_hbm, onehot_hbm, zeros_hbm, out_hbm,
                dstv, onehot, acc):
    c = lax.axis_index("c")
    s = lax.axis_index("s")
    wid = c * NS + s

    pltpu.sync_copy(onehot_hbm, onehot)
    pltpu.sync_copy(zeros_hbm, acc.at[pl.ds(s * 624, 624), :])

    @pl.when(s == NS - 1)
    def _():
        pltpu.sync_copy(zeros_hbm.at[pl.ds(0, 16), :],
                        acc.at[pl.ds(9984, 16), :])

    pltpu.sync_copy(dst_hbm.at[wid], dstv)
    plsc.subcore_barrier()

    def chunk(i, carry):
        pltpu.sync_copy(onehot, acc.at[dstv.at[i]], add=True)
        return carry
    lax.fori_loop(0, NCH, chunk, 0)

    plsc.subcore_barrier()
    _flush(acc, out_hbm, c, s)


@functools.partial(
    pl.kernel,
    out_type=jax.ShapeDtypeStruct((NC, N, D), jnp.float32),
    mesh=_mesh,
    scratch_types=[
        pltpu.VMEM((EPTA,), jnp.int32),
        pltpu.VMEM((NCHA, KA), jnp.int32),
        pltpu.VMEM((KA, D), jnp.float32),
        pltpu.VMEM((KA, D), jnp.float32),
        pltpu.VMEM_SHARED((N, D), jnp.float32),
        pltpu.SemaphoreType.DMA,
        pltpu.SemaphoreType.DMA,
    ],
)
def _agg_kernel(src_hbm, dst_hbm, feat_hbm, zeros_hbm, out_hbm,
                srcv, dstv, rows0, rows1, acc, sem0, sem1):
    c = lax.axis_index("c")
    s = lax.axis_index("s")
    wid = c * NS + s

    zb = s * 624
    pltpu.sync_copy(zeros_hbm, acc.at[pl.ds(zb, 624), :])

    @pl.when(s == NS - 1)
    def _():
        pltpu.sync_copy(zeros_hbm.at[pl.ds(0, 16), :],
                        acc.at[pl.ds(9984, 16), :])

    pltpu.sync_copy(src_hbm.at[pl.ds(wid * EPTA, EPTA)], srcv)
    pltpu.sync_copy(dst_hbm.at[wid], dstv)
    plsc.subcore_barrier()

    def _gather(i, buf, sem):
        return pltpu.make_async_copy(
            feat_hbm.at[srcv.at[pl.ds(i * KA, KA)]], buf, sem)

    _gather(0, rows0, sem0).start()

    def chunk2(j, carry):
        i0 = 2 * j
        _gather(i0 + 1, rows1, sem1).start()
        _gather(i0, rows0, sem0).wait()
        pltpu.sync_copy(rows0, acc.at[dstv.at[i0]], add=True)

        @pl.when(j < NCHA // 2 - 1)
        def _():
            _gather(i0 + 2, rows0, sem0).start()

        _gather(i0 + 1, rows1, sem1).wait()
        pltpu.sync_copy(rows1, acc.at[dstv.at[i0 + 1]], add=True)
        return carry
    lax.fori_loop(0, NCHA // 2, chunk2, 0)

    plsc.subcore_barrier()
    _flush(acc, out_hbm, c, s)


def _dense1_body(degp_ref, x_ref, w1_ref, dinv_ref, hs1_ref):
    deg = degp_ref[0] + degp_ref[1] + 1.0
    dinv = lax.rsqrt(deg)
    dinv_ref[...] = dinv
    h = jnp.dot(x_ref[...], w1_ref[...], preferred_element_type=jnp.float32)
    hs1_ref[pl.ds(0, N), :] = h * dinv
    hs1_ref[pl.ds(N, 8), :] = jnp.zeros((8, D), jnp.float32)


_dense1 = pl.pallas_call(
    _dense1_body,
    out_shape=(
        jax.ShapeDtypeStruct((N, 1), jnp.float32),
        jax.ShapeDtypeStruct((NP, D), jnp.float32),
    ),
)


def _dense2_body(aggp_ref, hs1_ref, dinv_ref, b1_ref, g_ref, bt_ref, w2_ref,
                 hs2_ref):
    dinv = dinv_ref[...]
    hs1 = hs1_ref[pl.ds(0, N), :]
    out1 = dinv * (aggp_ref[0] + aggp_ref[1] + hs1) + b1_ref[...]
    mean = jnp.mean(out1, axis=0, keepdims=True)
    ctr = out1 - mean
    var = jnp.mean(ctr * ctr, axis=0, keepdims=True)
    y = ctr * lax.rsqrt(var + 1e-5) * g_ref[...] + bt_ref[...]
    y = jnp.maximum(y, 0.0)
    h2 = jnp.dot(y, w2_ref[...], preferred_element_type=jnp.float32)
    hs2_ref[pl.ds(0, N), :] = h2 * dinv
    hs2_ref[pl.ds(N, 8), :] = jnp.zeros((8, D), jnp.float32)


_dense2 = pl.pallas_call(
    _dense2_body,
    out_shape=jax.ShapeDtypeStruct((NP, D), jnp.float32),
)


def _dense3_body(aggp_ref, hs2_ref, dinv_ref, b2_ref, out_ref):
    agg = aggp_ref[0] + aggp_ref[1] + hs2_ref[pl.ds(0, N), :]
    out_ref[...] = dinv_ref[...] * agg + b2_ref[...]


_dense3 = pl.pallas_call(
    _dense3_body,
    out_shape=jax.ShapeDtypeStruct((N, D), jnp.float32),
)


def kernel(x, edge_index, W1, b1, gamma, beta, W2, b2):
    pad = NW * EPTA - E
    src_flat = jnp.concatenate(
        [edge_index[0], jnp.full((pad,), N, jnp.int32)])
    dst3d = jnp.concatenate(
        [edge_index[1], jnp.zeros((pad,), jnp.int32)]).reshape(NW, NCHA, KA)
    dst3d_deg = edge_index[1].reshape(NW, NCH, K)
    zeros = jnp.zeros((624, D), jnp.float32)
    onehot = jnp.zeros((K, D), jnp.float32).at[:, 0].set(1.0)
    degp = _deg_kernel(dst3d_deg, onehot, zeros)
    degp_col = degp[:, :, 0:1]
    dinv, hs1 = _dense1(degp_col, x, W1)
    aggp1 = _agg_kernel(src_flat, dst3d, hs1, zeros)
    hs2 = _dense2(aggp1, hs1, dinv,
                  b1.reshape(1, D), gamma.reshape(1, D), beta.reshape(1, D),
                  W2)
    aggp2 = _agg_kernel(src_flat, dst3d, hs2, zeros)
    return _dense3(aggp2, hs2, dinv, b2.reshape(1, D))


# speedup vs baseline: 1.1774x; 1.1774x over previous
import functools

import jax
import jax.numpy as jnp
from jax import lax
from jax.experimental import pallas as pl
from jax.experimental.pallas import tpu as pltpu
from jax.experimental.pallas import tpu_sc as plsc

N = 10000
E = 320000
D = 128
NC = 2
NS = 16
NW = NC * NS
K = 125
EPT = E // NW
NCH = EPT // K
ROWS_PT = N // NS

_mesh = plsc.VectorSubcoreMesh(core_axis_name="c", subcore_axis_name="s")


def _flush(acc, out_hbm, c, s):
    fbase = s * 624
    pltpu.sync_copy(acc.at[pl.ds(fbase, 624), :],
                    out_hbm.at[c, pl.ds(fbase, 624), :])

    @pl.when(s == NS - 1)
    def _():
        pltpu.sync_copy(acc.at[pl.ds(9984, 16), :],
                        out_hbm.at[c, pl.ds(9984, 16), :])


@functools.partial(
    pl.kernel,
    out_type=jax.ShapeDtypeStruct((NC, N, D), jnp.float32),
    mesh=_mesh,
    scratch_types=[
        pltpu.VMEM((NCH, K), jnp.int32),
        pltpu.VMEM((K, D), jnp.float32),
        pltpu.VMEM_SHARED((N, D), jnp.float32),
    ],
)
def _deg_kernel(dst_hbm, onehot_hbm, zeros_hbm, out_hbm,
                dstv, onehot, acc):
    c = lax.axis_index("c")
    s = lax.axis_index("s")
    wid = c * NS + s

    pltpu.sync_copy(onehot_hbm, onehot)
    pltpu.sync_copy(zeros_hbm, acc.at[pl.ds(s * 624, 624), :])

    @pl.when(s == NS - 1)
    def _():
        pltpu.sync_copy(zeros_hbm.at[pl.ds(0, 16), :],
                        acc.at[pl.ds(9984, 16), :])

    pltpu.sync_copy(dst_hbm.at[wid], dstv)
    plsc.subcore_barrier()

    def chunk(i, carry):
        pltpu.sync_copy(onehot, acc.at[dstv.at[i]], add=True)
        return carry
    lax.fori_loop(0, NCH, chunk, 0)

    plsc.subcore_barrier()
    _flush(acc, out_hbm, c, s)


@functools.partial(
    pl.kernel,
    out_type=jax.ShapeDtypeStruct((NC, N, D), jnp.float32),
    mesh=_mesh,
    scratch_types=[
        pltpu.VMEM((NCH, K), jnp.int32),
        pltpu.VMEM((NCH, K), jnp.int32),
        pltpu.VMEM((K, D), jnp.float32),
        pltpu.VMEM_SHARED((N, D), jnp.float32),
        pltpu.SemaphoreType.DMA,
    ],
)
def _agg_kernel(src_hbm, dst_hbm, feat_hbm, zeros_hbm, out_hbm,
                srcv, dstv, rows, acc, gsem):
    c = lax.axis_index("c")
    s = lax.axis_index("s")
    wid = c * NS + s

    zb = s * 624
    pltpu.sync_copy(zeros_hbm, acc.at[pl.ds(zb, 624), :])

    @pl.when(s == NS - 1)
    def _():
        pltpu.sync_copy(zeros_hbm.at[pl.ds(0, 16), :],
                        acc.at[pl.ds(9984, 16), :])

    pltpu.sync_copy(src_hbm.at[wid], srcv)
    pltpu.sync_copy(dst_hbm.at[wid], dstv)
    plsc.subcore_barrier()

    def chunk(i, carry):
        pltpu.async_copy(feat_hbm.at[srcv.at[i]], rows, gsem).wait()
        pltpu.sync_copy(rows, acc.at[dstv.at[i]], add=True)
        return carry
    lax.fori_loop(0, NCH, chunk, 0)

    plsc.subcore_barrier()
    _flush(acc, out_hbm, c, s)


def _dense1_body(degp_ref, x_ref, w1_ref, dinv_ref, hs1_ref):
    deg = (degp_ref[0, :, 0:1] + degp_ref[1, :, 0:1]) + 1.0
    dinv = lax.rsqrt(deg)
    dinv_ref[...] = dinv
    h = jnp.dot(x_ref[...], w1_ref[...], preferred_element_type=jnp.float32)
    hs1_ref[...] = h * dinv


_dense1 = pl.pallas_call(
    _dense1_body,
    out_shape=(
        jax.ShapeDtypeStruct((N, 1), jnp.float32),
        jax.ShapeDtypeStruct((N, D), jnp.float32),
    ),
)


def _dense2_body(aggp_ref, hs1_ref, dinv_ref, b1_ref, g_ref, bt_ref, w2_ref,
                 hs2_ref):
    dinv = dinv_ref[...]
    out1 = dinv * (aggp_ref[0] + aggp_ref[1] + hs1_ref[...]) + b1_ref[...]
    mean = jnp.mean(out1, axis=0, keepdims=True)
    ctr = out1 - mean
    var = jnp.mean(ctr * ctr, axis=0, keepdims=True)
    y = ctr * lax.rsqrt(var + 1e-5) * g_ref[...] + bt_ref[...]
    y = jnp.maximum(y, 0.0)
    h2 = jnp.dot(y, w2_ref[...], preferred_element_type=jnp.float32)
    hs2_ref[...] = h2 * dinv


_dense2 = pl.pallas_call(
    _dense2_body,
    out_shape=jax.ShapeDtypeStruct((N, D), jnp.float32),
)


def _dense3_body(aggp_ref, hs2_ref, dinv_ref, b2_ref, out_ref):
    agg = aggp_ref[0] + aggp_ref[1] + hs2_ref[...]
    out_ref[...] = dinv_ref[...] * agg + b2_ref[...]


_dense3 = pl.pallas_call(
    _dense3_body,
    out_shape=jax.ShapeDtypeStruct((N, D), jnp.float32),
)


def kernel(x, edge_index, W1, b1, gamma, beta, W2, b2):
    src3d = edge_index[0].reshape(NW, NCH, K)
    dst3d = edge_index[1].reshape(NW, NCH, K)
    zeros = jnp.zeros((624, D), jnp.float32)
    onehot = jnp.zeros((K, D), jnp.float32).at[:, 0].set(1.0)
    degp = _deg_kernel(dst3d, onehot, zeros)
    dinv, hs1 = _dense1(degp, x, W1)
    aggp1 = _agg_kernel(src3d, dst3d, hs1, zeros)
    hs2 = _dense2(aggp1, hs1, dinv,
                  b1.reshape(1, D), gamma.reshape(1, D), beta.reshape(1, D),
                  W2)
    aggp2 = _agg_kernel(src3d, dst3d, hs2, zeros)
    return _dense3(aggp2, hs2, dinv, b2.reshape(1, D))
